# Initial kernel scaffold; baseline (speedup 1.0000x reference)
import jax
import jax.numpy as jnp
from jax.experimental import pallas as pl


def kernel(x, tables):
    raise NotImplementedError("write your pallas kernel here")


# speedup vs baseline: 1.1995x; 1.1995x over previous
import functools

import jax
import jax.numpy as jnp
from jax import lax
from jax.experimental import pallas as pl
from jax.experimental.pallas import tpu as pltpu
from jax.experimental.pallas import tpu_sc as plsc

N_FIELDS = 26
VOCAB = 100000
D_EMB = 32
BATCH = 16384

NC = 2
NS = 16
NW = NC * NS
LANES = 16

R = BATCH * N_FIELDS
RW = R // NW
CHUNK = 1024
NCHUNK = RW // CHUNK
SUB = 128
NSUB = CHUNK // SUB


def _sc_gather(x_flat, tab_flat):
    mesh = plsc.VectorSubcoreMesh(core_axis_name="c", subcore_axis_name="s")

    @functools.partial(
        pl.kernel,
        mesh=mesh,
        out_type=jax.ShapeDtypeStruct((R, D_EMB), jnp.float32),
        scratch_types=[
            pltpu.VMEM((CHUNK,), jnp.int32),
            pltpu.VMEM((CHUNK,), jnp.int32),
            pltpu.VMEM((CHUNK, D_EMB), jnp.float32),
            pltpu.SemaphoreType.DMA,
        ],
        compiler_params=pltpu.CompilerParams(use_tc_tiling_on_sc=False),
    )
    def k(x_hbm, tab_hbm, out_hbm, raw_v, idx_v, rows_v, sem):
        cid = lax.axis_index("c")
        sid = lax.axis_index("s")
        wid = sid * NC + cid
        base_w = wid * RW

        def chunk_body(ci, carry):
            base = pl.multiple_of(base_w + ci * CHUNK, CHUNK)
            pltpu.sync_copy(x_hbm.at[pl.ds(base, CHUNK)], raw_v)

            def vec_body(j, carry2):
                r0 = base + j * LANES
                rvec = lax.iota(jnp.int32, LANES) + r0
                off = lax.rem(rvec, N_FIELDS) * VOCAB
                raw = raw_v[pl.ds(j * LANES, LANES)]
                idx_v[pl.ds(j * LANES, LANES)] = raw + off
                return carry2

            lax.fori_loop(0, CHUNK // LANES, vec_body, 0, unroll=4)

            copies = []
            for s in range(NSUB):
                copies.append(
                    pltpu.async_copy(
                        tab_hbm.at[idx_v.at[pl.ds(s * SUB, SUB)]],
                        rows_v.at[pl.ds(s * SUB, SUB)],
                        sem,
                    )
                )
            for c in copies:
                c.wait()

            pltpu.sync_copy(rows_v, out_hbm.at[pl.ds(base, CHUNK)])
            return carry

        lax.fori_loop(0, NCHUNK, chunk_body, 0)

    return k(x_flat, tab_flat)


def kernel(x, tables):
    x_flat = x.reshape(-1).astype(jnp.int32)
    tab_flat = tables.reshape(N_FIELDS * VOCAB, D_EMB)
    out = _sc_gather(x_flat, tab_flat)
    return out.reshape(BATCH, N_FIELDS * D_EMB)


# speedup vs baseline: 1.2023x; 1.0023x over previous
import functools

import jax
import jax.numpy as jnp
from jax import lax
from jax.experimental import pallas as pl
from jax.experimental.pallas import tpu as pltpu
from jax.experimental.pallas import tpu_sc as plsc

N_FIELDS = 26
VOCAB = 100000
D_EMB = 32
BATCH = 16384

NC = 2
NS = 16
NW = NC * NS
LANES = 16

R = BATCH * N_FIELDS
RW = R // NW
CHUNK = 512
NCHUNK = RW // CHUNK
SUB = 128
NSUB = CHUNK // SUB
NPAIR = NCHUNK // 2


def _sc_gather(x_flat, tab_flat):
    mesh = plsc.VectorSubcoreMesh(core_axis_name="c", subcore_axis_name="s")

    @functools.partial(
        pl.kernel,
        mesh=mesh,
        out_type=jax.ShapeDtypeStruct((R, D_EMB), jnp.float32),
        scratch_types=[
            pltpu.VMEM((RW,), jnp.int32),
            pltpu.VMEM((RW,), jnp.int32),
            pltpu.VMEM((CHUNK, D_EMB), jnp.float32),
            pltpu.VMEM((CHUNK, D_EMB), jnp.float32),
            pltpu.SemaphoreType.DMA,
            pltpu.SemaphoreType.DMA,
            pltpu.SemaphoreType.DMA,
            pltpu.SemaphoreType.DMA,
        ],
        compiler_params=pltpu.CompilerParams(use_tc_tiling_on_sc=False),
    )
    def k(x_hbm, tab_hbm, out_hbm, raw_v, idx_v, rows0, rows1,
          gsem0, gsem1, wsem0, wsem1):
        cid = lax.axis_index("c")
        sid = lax.axis_index("s")
        wid = sid * NC + cid
        base_w = wid * RW

        pltpu.sync_copy(x_hbm.at[pl.ds(base_w, RW)], raw_v)

        def vec_body(j, carry):
            r0 = base_w + j * LANES
            rvec = lax.iota(jnp.int32, LANES) + r0
            off = lax.rem(rvec, N_FIELDS) * VOCAB
            raw = raw_v[pl.ds(j * LANES, LANES)]
            idx_v[pl.ds(j * LANES, LANES)] = raw + off
            return carry

        lax.fori_loop(0, RW // LANES, vec_body, 0, unroll=8)

        def fire_g(ci, rows, gsem):
            off = ci * CHUNK
            for s in range(NSUB):
                pltpu.async_copy(
                    tab_hbm.at[idx_v.at[pl.ds(off + s * SUB, SUB)]],
                    rows.at[pl.ds(s * SUB, SUB)],
                    gsem,
                )

        def wait_g(rows, gsem):
            pltpu.make_async_copy(out_hbm.at[pl.ds(0, CHUNK)], rows, gsem).wait()

        def fire_w(ci, rows, wsem):
            pltpu.async_copy(rows, out_hbm.at[pl.ds(base_w + ci * CHUNK, CHUNK)],
                             wsem)

        def wait_w(ci, rows, wsem):
            pltpu.make_async_copy(
                rows, out_hbm.at[pl.ds(base_w + ci * CHUNK, CHUNK)], wsem
            ).wait()

        fire_g(0, rows0, gsem0)

        def pair_body(p, carry):
            a = 2 * p
            b = a + 1
            wait_g(rows0, gsem0)
            fire_w(a, rows0, wsem0)

            @pl.when(p > 0)
            def _():
                wait_w(b - 2, rows1, wsem1)

            fire_g(b, rows1, gsem1)
            wait_g(rows1, gsem1)
            fire_w(b, rows1, wsem1)
            wait_w(a, rows0, wsem0)

            @pl.when(p < NPAIR - 1)
            def _():
                fire_g(a + 2, rows0, gsem0)

            return carry

        lax.fori_loop(0, NPAIR, pair_body, 0)
        wait_w(NCHUNK - 1, rows1, wsem1)

    return k(x_flat, tab_flat)


def kernel(x, tables):
    x_flat = x.reshape(-1).astype(jnp.int32)
    tab_flat = tables.reshape(N_FIELDS * VOCAB, D_EMB)
    out = _sc_gather(x_flat, tab_flat)
    return out.reshape(BATCH, N_FIELDS * D_EMB)


# speedup vs baseline: 1.2028x; 1.0004x over previous
import functools

import jax
import jax.numpy as jnp
from jax import lax
from jax.experimental import pallas as pl
from jax.experimental.pallas import tpu as pltpu
from jax.experimental.pallas import tpu_sc as plsc

N_FIELDS = 26
VOCAB = 100000
D_EMB = 32
BATCH = 16384

NC = 2
NS = 16
NW = NC * NS
LANES = 16

R = BATCH * N_FIELDS
RW = R // NW
CHUNK = 512
NCHUNK = RW // CHUNK
SUB = 512
NSUB = CHUNK // SUB
NPAIR = NCHUNK // 2


def _sc_gather(x_flat, tab_flat):
    mesh = plsc.VectorSubcoreMesh(core_axis_name="c", subcore_axis_name="s")

    @functools.partial(
        pl.kernel,
        mesh=mesh,
        out_type=jax.ShapeDtypeStruct((R, D_EMB), jnp.float32),
        scratch_types=[
            pltpu.VMEM((RW,), jnp.int32),
            pltpu.VMEM((RW,), jnp.int32),
            pltpu.VMEM((CHUNK, D_EMB), jnp.float32),
            pltpu.VMEM((CHUNK, D_EMB), jnp.float32),
            pltpu.SemaphoreType.DMA,
            pltpu.SemaphoreType.DMA,
            pltpu.SemaphoreType.DMA,
            pltpu.SemaphoreType.DMA,
        ],
        compiler_params=pltpu.CompilerParams(use_tc_tiling_on_sc=False),
    )
    def k(x_hbm, tab_hbm, out_hbm, raw_v, idx_v, rows0, rows1,
          gsem0, gsem1, wsem0, wsem1):
        cid = lax.axis_index("c")
        sid = lax.axis_index("s")
        wid = sid * NC + cid
        base_w = wid * RW

        pltpu.sync_copy(x_hbm.at[pl.ds(base_w, RW)], raw_v)

        def vec_body(j, carry):
            r0 = base_w + j * LANES
            rvec = lax.iota(jnp.int32, LANES) + r0
            off = lax.rem(rvec, N_FIELDS) * VOCAB
            raw = raw_v[pl.ds(j * LANES, LANES)]
            idx_v[pl.ds(j * LANES, LANES)] = raw + off
            return carry

        lax.fori_loop(0, RW // LANES, vec_body, 0, unroll=8)

        def fire_g(ci, rows, gsem):
            off = ci * CHUNK
            for s in range(NSUB):
                pltpu.async_copy(
                    tab_hbm.at[idx_v.at[pl.ds(off + s * SUB, SUB)]],
                    rows.at[pl.ds(s * SUB, SUB)],
                    gsem,
                )

        def wait_g(rows, gsem):
            pltpu.make_async_copy(out_hbm.at[pl.ds(0, CHUNK)], rows, gsem).wait()

        def fire_w(ci, rows, wsem):
            pltpu.async_copy(rows, out_hbm.at[pl.ds(base_w + ci * CHUNK, CHUNK)],
                             wsem)

        def wait_w(ci, rows, wsem):
            pltpu.make_async_copy(
                rows, out_hbm.at[pl.ds(base_w + ci * CHUNK, CHUNK)], wsem
            ).wait()

        fire_g(0, rows0, gsem0)

        def pair_body(p, carry):
            a = 2 * p
            b = a + 1
            wait_g(rows0, gsem0)
            fire_w(a, rows0, wsem0)

            @pl.when(p > 0)
            def _():
                wait_w(b - 2, rows1, wsem1)

            fire_g(b, rows1, gsem1)
            wait_g(rows1, gsem1)
            fire_w(b, rows1, wsem1)
            wait_w(a, rows0, wsem0)

            @pl.when(p < NPAIR - 1)
            def _():
                fire_g(a + 2, rows0, gsem0)

            return carry

        lax.fori_loop(0, NPAIR, pair_body, 0)
        wait_w(NCHUNK - 1, rows1, wsem1)

    return k(x_flat, tab_flat)


def kernel(x, tables):
    x_flat = x.reshape(-1).astype(jnp.int32)
    tab_flat = tables.reshape(N_FIELDS * VOCAB, D_EMB)
    out = _sc_gather(x_flat, tab_flat)
    return out.reshape(BATCH, N_FIELDS * D_EMB)


# speedup vs baseline: 1.2037x; 1.0008x over previous
import functools

import jax
import jax.numpy as jnp
from jax import lax
from jax.experimental import pallas as pl
from jax.experimental.pallas import tpu as pltpu
from jax.experimental.pallas import tpu_sc as plsc

N_FIELDS = 26
VOCAB = 100000
D_EMB = 32
BATCH = 16384

NC = 2
NS = 16
NW = NC * NS
LANES = 16

R = BATCH * N_FIELDS
RW = R // NW
CHUNK = 512
NCHUNK = RW // CHUNK
SUB = 512
NSUB = CHUNK // SUB
NPAIR = NCHUNK // 2


def _sc_gather(x_flat, tab_flat):
    mesh = plsc.VectorSubcoreMesh(core_axis_name="c", subcore_axis_name="s")

    @functools.partial(
        pl.kernel,
        mesh=mesh,
        out_type=jax.ShapeDtypeStruct((R, D_EMB), jnp.float32),
        scratch_types=[
            pltpu.VMEM((RW,), jnp.int32),
            pltpu.VMEM((RW,), jnp.int32),
            pltpu.VMEM((CHUNK, D_EMB), jnp.float32),
            pltpu.VMEM((CHUNK, D_EMB), jnp.float32),
            pltpu.SemaphoreType.DMA,
            pltpu.SemaphoreType.DMA,
            pltpu.SemaphoreType.DMA,
            pltpu.SemaphoreType.DMA,
        ],
        compiler_params=pltpu.CompilerParams(use_tc_tiling_on_sc=False),
    )
    def k(x_hbm, tab_hbm, out_hbm, raw_v, idx_v, rows0, rows1,
          gsem0, gsem1, wsem0, wsem1):
        cid = lax.axis_index("c")
        sid = lax.axis_index("s")
        wid = sid * NC + cid
        base_w = wid * RW

        pltpu.sync_copy(x_hbm.at[pl.ds(base_w, RW)], raw_v)

        def vec_body(j, carry):
            r0 = base_w + j * LANES
            rvec = lax.iota(jnp.int32, LANES) + r0
            off = lax.rem(rvec, N_FIELDS) * VOCAB
            raw = raw_v[pl.ds(j * LANES, LANES)]
            idx_v[pl.ds(j * LANES, LANES)] = raw + off
            return carry

        lax.fori_loop(0, RW // LANES, vec_body, 0, unroll=8)

        def fire_g(ci, rows, gsem):
            off = ci * CHUNK

            def fire_one(g, carry):
                idx = idx_v[pl.ds(off + g * LANES, LANES)]
                pltpu.async_copy(
                    tab_hbm.at[idx],
                    rows.at[pl.ds(g * LANES, LANES)],
                    gsem,
                )
                return carry

            lax.fori_loop(0, CHUNK // LANES, fire_one, 0, unroll=4)

        def wait_g(rows, gsem):
            pltpu.make_async_copy(out_hbm.at[pl.ds(0, CHUNK)], rows, gsem).wait()

        def fire_w(ci, rows, wsem):
            pltpu.async_copy(rows, out_hbm.at[pl.ds(base_w + ci * CHUNK, CHUNK)],
                             wsem)

        def wait_w(ci, rows, wsem):
            pltpu.make_async_copy(
                rows, out_hbm.at[pl.ds(base_w + ci * CHUNK, CHUNK)], wsem
            ).wait()

        fire_g(0, rows0, gsem0)

        def pair_body(p, carry):
            a = 2 * p
            b = a + 1
            wait_g(rows0, gsem0)
            fire_w(a, rows0, wsem0)

            @pl.when(p > 0)
            def _():
                wait_w(b - 2, rows1, wsem1)

            fire_g(b, rows1, gsem1)
            wait_g(rows1, gsem1)
            fire_w(b, rows1, wsem1)
            wait_w(a, rows0, wsem0)

            @pl.when(p < NPAIR - 1)
            def _():
                fire_g(a + 2, rows0, gsem0)

            return carry

        lax.fori_loop(0, NPAIR, pair_body, 0)
        wait_w(NCHUNK - 1, rows1, wsem1)

    return k(x_flat, tab_flat)


def kernel(x, tables):
    x_flat = x.reshape(-1).astype(jnp.int32)
    tab_flat = tables.reshape(N_FIELDS * VOCAB, D_EMB)
    out = _sc_gather(x_flat, tab_flat)
    return out.reshape(BATCH, N_FIELDS * D_EMB)


# speedup vs baseline: 4.1261x; 3.4279x over previous
import functools

import jax
import jax.numpy as jnp
from jax import lax
from jax.experimental import pallas as pl
from jax.experimental.pallas import tpu as pltpu
from jax.experimental.pallas import tpu_sc as plsc

N_FIELDS = 26
VOCAB = 100000
D_EMB = 32
BATCH = 16384

NC = 2
NS = 16
NW = NC * NS
LANES = 16

NU = N_FIELDS * D_EMB
UPW = NU // NW
QB = 4096
NQ = BATCH // QB


def _sc_lookup(x_t, tab2):
    mesh = plsc.VectorSubcoreMesh(core_axis_name="c", subcore_axis_name="s")

    @functools.partial(
        pl.kernel,
        mesh=mesh,
        out_type=jax.ShapeDtypeStruct((NU, BATCH), jnp.float32),
        scratch_types=[
            pltpu.VMEM((VOCAB,), jnp.float32),
            pltpu.VMEM((QB,), jnp.int32),
            pltpu.VMEM((QB,), jnp.int32),
            pltpu.VMEM((QB,), jnp.float32),
            pltpu.VMEM((QB,), jnp.float32),
            pltpu.SemaphoreType.DMA,
            pltpu.SemaphoreType.DMA,
            pltpu.SemaphoreType.DMA,
            pltpu.SemaphoreType.DMA,
            pltpu.SemaphoreType.DMA,
        ],
        compiler_params=pltpu.CompilerParams(
            use_tc_tiling_on_sc=True, needs_layout_passes=False),
    )
    def k(x_hbm, tab_hbm, out_hbm, vec, xq0, xq1, oq0, oq1,
          vsem, xsem0, xsem1, osem0, osem1):
        cid = lax.axis_index("c")
        sid = lax.axis_index("s")
        wid = sid * NC + cid
        u0 = wid * UPW

        def fire_x(u, q, xq, xsem):
            f = u // D_EMB
            pltpu.async_copy(x_hbm.at[f, pl.ds(q * QB, QB)], xq, xsem)

        def wait_x(xq, xsem):
            pltpu.make_async_copy(x_hbm.at[0, pl.ds(0, QB)], xq, xsem).wait()

        def fire_o(u, q, oq, osem):
            pltpu.async_copy(oq, out_hbm.at[u, pl.ds(q * QB, QB)], osem)

        def wait_o(oq, osem):
            pltpu.make_async_copy(oq, out_hbm.at[0, pl.ds(0, QB)], osem).wait()

        def gather_q(xq, oq):
            def body(g, carry):
                idx = xq[pl.ds(g * LANES, LANES)]
                oq[pl.ds(g * LANES, LANES)] = plsc.load_gather(vec, [idx])
                return carry

            lax.fori_loop(0, QB // LANES, body, 0, unroll=8)

        def unit_body(j, carry):
            u = u0 + j
            pltpu.async_copy(tab_hbm.at[u], vec, vsem)
            fire_x(u, 0, xq0, xsem0)
            pltpu.make_async_copy(tab_hbm.at[0], vec, vsem).wait()

            fire_x(u, 1, xq1, xsem1)
            wait_x(xq0, xsem0)
            gather_q(xq0, oq0)
            fire_o(u, 0, oq0, osem0)

            fire_x(u, 2, xq0, xsem0)
            wait_x(xq1, xsem1)
            gather_q(xq1, oq1)
            fire_o(u, 1, oq1, osem1)

            fire_x(u, 3, xq1, xsem1)
            wait_x(xq0, xsem0)
            wait_o(oq0, osem0)
            gather_q(xq0, oq0)
            fire_o(u, 2, oq0, osem0)

            wait_x(xq1, xsem1)
            wait_o(oq1, osem1)
            gather_q(xq1, oq1)
            fire_o(u, 3, oq1, osem1)

            wait_o(oq0, osem0)
            wait_o(oq1, osem1)
            return carry

        lax.fori_loop(0, UPW, unit_body, 0)

    return k(x_t, tab2)


def kernel(x, tables):
    x_t = x.T.astype(jnp.int32)
    tab2 = tables.transpose(0, 2, 1).reshape(NU, VOCAB)
    out_t = _sc_lookup(x_t, tab2)
    return out_t.T.reshape(BATCH, NU)


# speedup vs baseline: 4.1954x; 1.0168x over previous
import functools

import jax
import jax.numpy as jnp
from jax import lax
from jax.experimental import pallas as pl
from jax.experimental.pallas import tpu as pltpu
from jax.experimental.pallas import tpu_sc as plsc

N_FIELDS = 26
VOCAB = 100000
D_EMB = 32
BATCH = 16384

NC = 2
NS = 16
NW = NC * NS
LANES = 16

NU = N_FIELDS * D_EMB
UPW = NU // NW
QB = 4096
NQ = BATCH // QB


def _sc_lookup(x_t, tab2):
    mesh = plsc.VectorSubcoreMesh(core_axis_name="c", subcore_axis_name="s")

    @functools.partial(
        pl.kernel,
        mesh=mesh,
        out_type=jax.ShapeDtypeStruct((NU, BATCH), jnp.float32),
        scratch_types=[
            pltpu.VMEM((VOCAB,), jnp.float32),
            pltpu.VMEM((QB,), jnp.int32),
            pltpu.VMEM((QB,), jnp.int32),
            pltpu.VMEM((QB,), jnp.float32),
            pltpu.VMEM((QB,), jnp.float32),
            pltpu.SemaphoreType.DMA,
            pltpu.SemaphoreType.DMA,
            pltpu.SemaphoreType.DMA,
            pltpu.SemaphoreType.DMA,
            pltpu.SemaphoreType.DMA,
        ],
        compiler_params=pltpu.CompilerParams(
            use_tc_tiling_on_sc=True, needs_layout_passes=False),
    )
    def k(x_hbm, tab_hbm, out_hbm, vec, xq0, xq1, oq0, oq1,
          vsem, xsem0, xsem1, osem0, osem1):
        cid = lax.axis_index("c")
        sid = lax.axis_index("s")
        wid = sid * NC + cid
        u0 = wid * UPW

        def fire_x(u, q, xq, xsem):
            f = u // D_EMB
            pltpu.async_copy(x_hbm.at[f, pl.ds(q * QB, QB)], xq, xsem)

        def wait_x(xq, xsem):
            pltpu.make_async_copy(x_hbm.at[0, pl.ds(0, QB)], xq, xsem).wait()

        def fire_o(u, q, oq, osem):
            pltpu.async_copy(oq, out_hbm.at[u, pl.ds(q * QB, QB)], osem)

        def wait_o(oq, osem):
            pltpu.make_async_copy(oq, out_hbm.at[0, pl.ds(0, QB)], osem).wait()

        def gather_q(xq, oq):
            def body(g, carry):
                idx = xq[pl.ds(g * LANES, LANES)]
                oq[pl.ds(g * LANES, LANES)] = plsc.load_gather(vec, [idx])
                return carry

            lax.fori_loop(0, QB // LANES, body, 0, unroll=16)

        def unit_body(j, carry):
            u = u0 + j
            pltpu.async_copy(tab_hbm.at[u], vec, vsem)
            fire_x(u, 0, xq0, xsem0)
            pltpu.make_async_copy(tab_hbm.at[0], vec, vsem).wait()

            fire_x(u, 1, xq1, xsem1)
            wait_x(xq0, xsem0)

            @pl.when(j > 0)
            def _():
                wait_o(oq0, osem0)

            gather_q(xq0, oq0)
            fire_o(u, 0, oq0, osem0)

            fire_x(u, 2, xq0, xsem0)
            wait_x(xq1, xsem1)

            @pl.when(j > 0)
            def _():
                wait_o(oq1, osem1)

            gather_q(xq1, oq1)
            fire_o(u, 1, oq1, osem1)

            fire_x(u, 3, xq1, xsem1)
            wait_x(xq0, xsem0)
            wait_o(oq0, osem0)
            gather_q(xq0, oq0)
            fire_o(u, 2, oq0, osem0)

            wait_x(xq1, xsem1)
            wait_o(oq1, osem1)
            gather_q(xq1, oq1)
            fire_o(u, 3, oq1, osem1)
            return carry

        lax.fori_loop(0, UPW, unit_body, 0)
        wait_o(oq0, osem0)
        wait_o(oq1, osem1)

    return k(x_t, tab2)


def kernel(x, tables):
    x_t = x.T.astype(jnp.int32)
    tab2 = tables.transpose(0, 2, 1).reshape(NU, VOCAB)
    out_t = _sc_lookup(x_t, tab2)
    return out_t.T.reshape(BATCH, NU)


# speedup vs baseline: 4.2132x; 1.0042x over previous
import functools

import jax
import jax.numpy as jnp
from jax import lax
from jax.experimental import pallas as pl
from jax.experimental.pallas import tpu as pltpu
from jax.experimental.pallas import tpu_sc as plsc

N_FIELDS = 26
VOCAB = 100000
D_EMB = 32
BATCH = 16384

NC = 2
NS = 16
NW = NC * NS
LANES = 16

NU = N_FIELDS * D_EMB
UPW = NU // NW
QB = 4096
NQ = BATCH // QB


def _sc_lookup(x_t, tab2):
    mesh = plsc.VectorSubcoreMesh(core_axis_name="c", subcore_axis_name="s")

    @functools.partial(
        pl.kernel,
        mesh=mesh,
        out_type=jax.ShapeDtypeStruct((NU, BATCH), jnp.float32),
        scratch_types=[
            pltpu.VMEM((VOCAB,), jnp.float32),
            pltpu.VMEM((BATCH,), jnp.int32),
            pltpu.VMEM((QB,), jnp.float32),
            pltpu.VMEM((QB,), jnp.float32),
            pltpu.SemaphoreType.DMA,
            pltpu.SemaphoreType.DMA,
            pltpu.SemaphoreType.DMA,
        ],
        compiler_params=pltpu.CompilerParams(
            use_tc_tiling_on_sc=True, needs_layout_passes=False),
    )
    def k(x_hbm, tab_hbm, out_hbm, vec, xrow, oq0, oq1, vsem, osem0, osem1):
        cid = lax.axis_index("c")
        sid = lax.axis_index("s")
        wid = sid * NC + cid
        u0 = wid * UPW

        def fire_o(u, q, oq, osem):
            pltpu.async_copy(oq, out_hbm.at[u, pl.ds(q * QB, QB)], osem)

        def wait_o(oq, osem):
            pltpu.make_async_copy(oq, out_hbm.at[0, pl.ds(0, QB)], osem).wait()

        def gather_q(q, oq):
            def body(g, carry):
                idx = xrow[pl.ds(q * QB + g * LANES, LANES)]
                oq[pl.ds(g * LANES, LANES)] = plsc.load_gather(vec, [idx])
                return carry

            lax.fori_loop(0, QB // LANES, body, 0, unroll=16)

        def unit_body(j, carry):
            u = u0 + j
            pltpu.async_copy(tab_hbm.at[u], vec, vsem)

            @pl.when((j == 0) | (u % D_EMB == 0))
            def _():
                pltpu.sync_copy(x_hbm.at[u // D_EMB], xrow)

            pltpu.make_async_copy(tab_hbm.at[0], vec, vsem).wait()

            @pl.when(j > 0)
            def _():
                wait_o(oq0, osem0)

            gather_q(0, oq0)
            fire_o(u, 0, oq0, osem0)

            @pl.when(j > 0)
            def _():
                wait_o(oq1, osem1)

            gather_q(1, oq1)
            fire_o(u, 1, oq1, osem1)

            wait_o(oq0, osem0)
            gather_q(2, oq0)
            fire_o(u, 2, oq0, osem0)

            wait_o(oq1, osem1)
            gather_q(3, oq1)
            fire_o(u, 3, oq1, osem1)
            return carry

        lax.fori_loop(0, UPW, unit_body, 0)
        wait_o(oq0, osem0)
        wait_o(oq1, osem1)

    return k(x_t, tab2)


def kernel(x, tables):
    x_t = x.T.astype(jnp.int32)
    tab2 = tables.transpose(0, 2, 1).reshape(NU, VOCAB)
    out_t = _sc_lookup(x_t, tab2)
    return out_t.T.reshape(BATCH, NU)


# speedup vs baseline: 4.2169x; 1.0009x over previous
import functools

import jax
import jax.numpy as jnp
from jax import lax
from jax.experimental import pallas as pl
from jax.experimental.pallas import tpu as pltpu
from jax.experimental.pallas import tpu_sc as plsc

N_FIELDS = 26
VOCAB = 100000
D_EMB = 32
BATCH = 16384

NC = 2
NS = 16
NW = NC * NS
LANES = 16

NU = N_FIELDS * D_EMB
UPW = NU // NW
QB = 4096
NQ = BATCH // QB


def _sc_lookup(x_t, tab2):
    mesh = plsc.VectorSubcoreMesh(core_axis_name="c", subcore_axis_name="s")

    @functools.partial(
        pl.kernel,
        mesh=mesh,
        out_type=jax.ShapeDtypeStruct((NU, BATCH), jnp.float32),
        scratch_types=[
            pltpu.VMEM((VOCAB,), jnp.float32),
            pltpu.VMEM((BATCH,), jnp.int32),
            pltpu.VMEM((QB,), jnp.float32),
            pltpu.VMEM((QB,), jnp.float32),
            pltpu.SemaphoreType.DMA,
            pltpu.SemaphoreType.DMA,
            pltpu.SemaphoreType.DMA,
        ],
        compiler_params=pltpu.CompilerParams(
            use_tc_tiling_on_sc=True, needs_layout_passes=False),
    )
    def k(x_hbm, tab_hbm, out_hbm, vec, xrow, oq0, oq1, vsem, osem0, osem1):
        cid = lax.axis_index("c")
        sid = lax.axis_index("s")
        wid = sid * NC + cid
        u0 = wid * UPW

        def fire_o(u, q, oq, osem):
            pltpu.async_copy(oq, out_hbm.at[u, pl.ds(q * QB, QB)], osem)

        def wait_o(oq, osem):
            pltpu.make_async_copy(oq, out_hbm.at[0, pl.ds(0, QB)], osem).wait()

        def gather_q(q, oq):
            def body(g, carry):
                idx = xrow[pl.ds(q * QB + g * LANES, LANES)]
                oq[pl.ds(g * LANES, LANES)] = plsc.load_gather(vec, [idx])
                return carry

            lax.fori_loop(0, QB // LANES, body, 0, unroll=32)

        def unit_body(j, carry):
            u = u0 + j
            pltpu.async_copy(tab_hbm.at[u], vec, vsem)

            @pl.when((j == 0) | (u % D_EMB == 0))
            def _():
                pltpu.sync_copy(x_hbm.at[u // D_EMB], xrow)

            pltpu.make_async_copy(tab_hbm.at[0], vec, vsem).wait()

            @pl.when(j > 0)
            def _():
                wait_o(oq0, osem0)

            gather_q(0, oq0)
            fire_o(u, 0, oq0, osem0)

            @pl.when(j > 0)
            def _():
                wait_o(oq1, osem1)

            gather_q(1, oq1)
            fire_o(u, 1, oq1, osem1)

            wait_o(oq0, osem0)
            gather_q(2, oq0)
            fire_o(u, 2, oq0, osem0)

            wait_o(oq1, osem1)
            gather_q(3, oq1)
            fire_o(u, 3, oq1, osem1)
            return carry

        lax.fori_loop(0, UPW, unit_body, 0)
        wait_o(oq0, osem0)
        wait_o(oq1, osem1)

    return k(x_t, tab2)


def kernel(x, tables):
    x_t = x.T.astype(jnp.int32)
    tab2 = tables.transpose(0, 2, 1).reshape(NU, VOCAB)
    out_t = _sc_lookup(x_t, tab2)
    return out_t.T.reshape(BATCH, NU)
